# baseline (speedup 1.0000x reference)
import functools

import jax
import jax.numpy as jnp
from jax import lax
from jax.experimental import pallas as pl
from jax.experimental.pallas import tpu as pltpu
from jax.experimental.pallas import tpu_sc as plsc

B = 4
N = 1024
BN = B * N
KNN = 20
EPS = 1e-5
NW = 32
BF16 = jnp.bfloat16


def _topk_body(h_blk_ref, ht_ref, xxc_ref, xxr_ref, idx_ref, *, nb):
    b = pl.program_id(0)
    hb = h_blk_ref[0]
    ht = ht_ref[0]
    d = 2.0 * jnp.dot(hb.astype(BF16), ht.astype(BF16),
                      preferred_element_type=jnp.float32)
    xb = xxc_ref[0]
    xa = xxr_ref[0]
    d = d - xb - xa
    iota_m = lax.broadcasted_iota(jnp.int32, d.shape, 1)
    iota_k = lax.broadcasted_iota(jnp.int32, (nb, KNN), 1)
    out = jnp.zeros((nb, KNN), jnp.int32)
    for j in range(KNN):
        am = jnp.argmax(d, axis=1).astype(jnp.int32)[:, None]
        out = jnp.where(iota_k == j, am, out)
        d = jnp.where(iota_m == am, -jnp.inf, d)
    idx_ref[0] = jnp.transpose(out) + b * N


def _topk(h3d, ht3d, xx, c):
    nb = 1024
    body = functools.partial(_topk_body, nb=nb)
    return pl.pallas_call(
        body,
        grid=(B, N // nb),
        in_specs=[
            pl.BlockSpec((1, nb, c), lambda b, i: (b, i, 0)),
            pl.BlockSpec((1, c, N), lambda b, i: (b, 0, 0)),
            pl.BlockSpec((1, nb, 1), lambda b, i: (b, i, 0)),
            pl.BlockSpec((1, 1, N), lambda b, i: (b, 0, 0)),
        ],
        out_specs=pl.BlockSpec((1, KNN, nb), lambda b, i: (b, 0, i)),
        out_shape=jax.ShapeDtypeStruct((B, KNN, N), jnp.int32),
    )(h3d, ht3d, xx.reshape(B, N, 1), xx.reshape(B, 1, N))


def _sc_gather_rows(h2d, idx_flat, c):
    tot = KNN * BN
    rw = tot // NW
    ch = 128
    npair = rw // (2 * ch)
    mesh = plsc.VectorSubcoreMesh(core_axis_name="c", subcore_axis_name="s")

    @functools.partial(
        pl.kernel,
        mesh=mesh,
        compiler_params=pltpu.CompilerParams(use_tc_tiling_on_sc=False),
        out_type=jax.ShapeDtypeStruct((tot, c), jnp.float32),
        scratch_types=[
            pltpu.VMEM((ch,), jnp.int32),
            pltpu.VMEM((ch,), jnp.int32),
            pltpu.VMEM((ch, c), jnp.float32),
            pltpu.VMEM((ch, c), jnp.float32),
            pltpu.SemaphoreType.DMA,
            pltpu.SemaphoreType.DMA,
            pltpu.SemaphoreType.DMA,
            pltpu.SemaphoreType.DMA,
        ],
    )
    def sck(h_hbm, idx_hbm, out_hbm, idx0, idx1, rows0, rows1,
            sem_g, sem_g2, sem_o0, sem_o1):
        wid = lax.axis_index("s") * 2 + lax.axis_index("c")
        base = wid * rw

        def pair(p, carry):
            off0 = base + (2 * p) * ch
            off1 = off0 + ch

            @pl.when(p >= 1)
            def _():
                pltpu.make_async_copy(
                    rows0, out_hbm.at[pl.ds(off0 - 2 * ch, ch)], sem_o0).wait()

            pltpu.sync_copy(idx_hbm.at[pl.ds(off0, ch)], idx0)
            pltpu.async_copy(h_hbm.at[idx0], rows0, sem_g)

            @pl.when(p >= 1)
            def _():
                pltpu.make_async_copy(
                    rows1, out_hbm.at[pl.ds(off1 - 2 * ch, ch)], sem_o1).wait()

            pltpu.sync_copy(idx_hbm.at[pl.ds(off1, ch)], idx1)
            pltpu.async_copy(h_hbm.at[idx1], rows1, sem_g2)
            pltpu.make_async_copy(h_hbm.at[idx0], rows0, sem_g).wait()
            pltpu.async_copy(rows0, out_hbm.at[pl.ds(off0, ch)], sem_o0)
            pltpu.make_async_copy(h_hbm.at[idx1], rows1, sem_g2).wait()
            pltpu.async_copy(rows1, out_hbm.at[pl.ds(off1, ch)], sem_o1)
            return carry

        lax.fori_loop(0, npair, pair, 0, unroll=False)
        end0 = base + (2 * npair - 2) * ch
        pltpu.make_async_copy(rows0, out_hbm.at[pl.ds(end0, ch)], sem_o0).wait()
        pltpu.make_async_copy(rows1, out_hbm.at[pl.ds(end0 + ch, ch)], sem_o1).wait()

    return sck(h2d, idx_flat)


def _econv_body(nbr_ref, h_ref, wa_ref, wb_ref, b_ref, mx_ref, t_ref, sums_ref):
    b = pl.program_id(0)
    i = pl.program_id(1)
    hb = h_ref[0]
    wab = wa_ref[...].astype(BF16)
    s1 = None
    for j in range(KNN):
        diff = nbr_ref[0, j] - hb
        aj = jnp.dot(diff.astype(BF16), wab, preferred_element_type=jnp.float32)
        if s1 is None:
            s1, s2, mx = aj, aj * aj, aj
        else:
            s1 = s1 + aj
            s2 = s2 + aj * aj
            mx = jnp.maximum(mx, aj)
    t = jnp.dot(hb.astype(BF16), wb_ref[...].astype(BF16),
                preferred_element_type=jnp.float32) + b_ref[...]
    mx_ref[0] = mx
    t_ref[0] = t

    @pl.when(jnp.logical_and(b == 0, i == 0))
    def _():
        sums_ref[...] = jnp.zeros_like(sums_ref)

    kf = float(KNN)
    sy = jnp.sum(s1 + kf * t, axis=0)
    sy2 = jnp.sum(s2 + 2.0 * t * s1 + kf * t * t, axis=0)
    sums_ref[0:1, :] += sy[None, :]
    sums_ref[1:2, :] += sy2[None, :]


def _econv(nbr4, h3d, wa, wb, bias, c, cout):
    nb = 256
    return pl.pallas_call(
        _econv_body,
        grid=(B, N // nb),
        in_specs=[
            pl.BlockSpec((1, KNN, nb, c), lambda b, i: (b, 0, i, 0)),
            pl.BlockSpec((1, nb, c), lambda b, i: (b, i, 0)),
            pl.BlockSpec((c, cout), lambda b, i: (0, 0)),
            pl.BlockSpec((c, cout), lambda b, i: (0, 0)),
            pl.BlockSpec((1, cout), lambda b, i: (0, 0)),
        ],
        out_specs=[
            pl.BlockSpec((1, nb, cout), lambda b, i: (b, i, 0)),
            pl.BlockSpec((1, nb, cout), lambda b, i: (b, i, 0)),
            pl.BlockSpec((8, cout), lambda b, i: (0, 0)),
        ],
        out_shape=[
            jax.ShapeDtypeStruct((B, N, cout), jnp.float32),
            jax.ShapeDtypeStruct((B, N, cout), jnp.float32),
            jax.ShapeDtypeStruct((8, cout), jnp.float32),
        ],
    )(nbr4, h3d, wa, wb, bias)


def _apply_body(mx_ref, t_ref, sums_ref, g_ref, be_ref, out_ref, *, count):
    inv_cnt = 1.0 / count
    m = sums_ref[0:1, :] * inv_cnt
    ey2 = sums_ref[1:2, :] * inv_cnt
    v = ey2 - m * m
    denom = jnp.sqrt(v + EPS)
    y = mx_ref[...] + t_ref[...]
    out_ref[...] = jnp.maximum(((y - m) / denom) * g_ref[...] + be_ref[...], 0.0)


def _apply(mx, t, sums, g, be, cout, count):
    mb = 1024
    body = functools.partial(_apply_body, count=count)
    return pl.pallas_call(
        body,
        grid=(BN // mb,),
        in_specs=[
            pl.BlockSpec((mb, cout), lambda i: (i, 0)),
            pl.BlockSpec((mb, cout), lambda i: (i, 0)),
            pl.BlockSpec((8, cout), lambda i: (0, 0)),
            pl.BlockSpec((1, cout), lambda i: (0, 0)),
            pl.BlockSpec((1, cout), lambda i: (0, 0)),
        ],
        out_specs=pl.BlockSpec((mb, cout), lambda i: (i, 0)),
        out_shape=jax.ShapeDtypeStruct((BN, cout), jnp.float32),
    )(mx, t, sums, g, be)


def _final_mm_body(h1_ref, h2_ref, mx3_ref, t3_ref, sums3_ref, g3_ref, be3_ref,
                   w1_ref, w2_ref, w3_ref, b_ref, f_ref, sums_ref, *, count3):
    i = pl.program_id(0)
    inv_cnt = 1.0 / count3
    m3 = sums3_ref[0:1, :] * inv_cnt
    v3 = sums3_ref[1:2, :] * inv_cnt - m3 * m3
    y3 = mx3_ref[...] + t3_ref[...]
    h3 = jnp.maximum(((y3 - m3) / jnp.sqrt(v3 + EPS)) * g3_ref[...]
                     + be3_ref[...], 0.0)
    f = jnp.dot(h1_ref[...].astype(BF16), w1_ref[...].astype(BF16),
                preferred_element_type=jnp.float32)
    f += jnp.dot(h2_ref[...].astype(BF16), w2_ref[...].astype(BF16),
                 preferred_element_type=jnp.float32)
    f += jnp.dot(h3.astype(BF16), w3_ref[...].astype(BF16),
                 preferred_element_type=jnp.float32)
    f = f + b_ref[...]
    f_ref[...] = f

    @pl.when(i == 0)
    def _():
        sums_ref[...] = jnp.zeros_like(sums_ref)

    sums_ref[0:1, :] += jnp.sum(f, axis=0)[None, :]
    sums_ref[1:2, :] += jnp.sum(f * f, axis=0)[None, :]


def _final_mm(h1, h2, mx3, t3, sums3, g3, be3, w1, w2, w3, bias):
    mb = 1024
    body = functools.partial(_final_mm_body, count3=BN * KNN)
    return pl.pallas_call(
        body,
        grid=(BN // mb,),
        in_specs=[
            pl.BlockSpec((mb, 64), lambda i: (i, 0)),
            pl.BlockSpec((mb, 128), lambda i: (i, 0)),
            pl.BlockSpec((mb, 256), lambda i: (i, 0)),
            pl.BlockSpec((mb, 256), lambda i: (i, 0)),
            pl.BlockSpec((8, 256), lambda i: (0, 0)),
            pl.BlockSpec((1, 256), lambda i: (0, 0)),
            pl.BlockSpec((1, 256), lambda i: (0, 0)),
            pl.BlockSpec((64, 512), lambda i: (0, 0)),
            pl.BlockSpec((128, 512), lambda i: (0, 0)),
            pl.BlockSpec((256, 512), lambda i: (0, 0)),
            pl.BlockSpec((1, 512), lambda i: (0, 0)),
        ],
        out_specs=[
            pl.BlockSpec((mb, 512), lambda i: (i, 0)),
            pl.BlockSpec((8, 512), lambda i: (0, 0)),
        ],
        out_shape=[
            jax.ShapeDtypeStruct((BN, 512), jnp.float32),
            jax.ShapeDtypeStruct((8, 512), jnp.float32),
        ],
    )(h1, h2, mx3, t3, sums3, g3, be3, w1, w2, w3, bias)


def _final_apply(f, sums, g, be, count):
    nb = 256

    def body(f_ref, sums_ref, g_ref, be_ref, out_ref):
        inv_cnt = 1.0 / count
        m = sums_ref[0:1, :] * inv_cnt
        ey2 = sums_ref[1:2, :] * inv_cnt
        v = ey2 - m * m
        denom = jnp.sqrt(v + EPS)
        y = jnp.maximum(((f_ref[0] - m) / denom) * g_ref[...] + be_ref[...], 0.0)
        out_ref[0] = jnp.transpose(y)

    return pl.pallas_call(
        body,
        grid=(B, N // nb),
        in_specs=[
            pl.BlockSpec((1, nb, 512), lambda b, i: (b, i, 0)),
            pl.BlockSpec((8, 512), lambda b, i: (0, 0)),
            pl.BlockSpec((1, 512), lambda b, i: (0, 0)),
            pl.BlockSpec((1, 512), lambda b, i: (0, 0)),
        ],
        out_specs=pl.BlockSpec((1, 512, nb), lambda b, i: (b, 0, i)),
        out_shape=jax.ShapeDtypeStruct((B, 512, N), jnp.float32),
    )(f.reshape(B, N, 512), sums, g, be)


def _edge_layer(h2d, c, cout, w, bias, g, be, apply_now=True):
    wa = jnp.transpose(w[:, :c])
    wb = jnp.transpose(w[:, c:])
    cpad = max(c, 16)
    if cpad != c:
        pad = ((0, cpad - c), (0, 0))
        h2d_p = jnp.pad(h2d, ((0, 0), (0, cpad - c)))
        wa = jnp.pad(wa, pad)
        wb = jnp.pad(wb, pad)
    else:
        h2d_p = h2d
    h3d = h2d_p.reshape(B, N, cpad)
    ht3d = jnp.transpose(h3d, (0, 2, 1))
    xx = jnp.sum(ht3d * ht3d, axis=1)
    gidx = _topk(h3d, ht3d, xx, cpad)
    idx_flat = gidx.reshape(B * KNN * N)
    nbr = _sc_gather_rows(h2d_p, idx_flat, cpad)
    nbr4 = nbr.reshape(B, KNN, N, cpad)
    mx, t3, sums = _econv(nbr4, h3d, wa, wb, bias[None, :], cpad, cout)
    if not apply_now:
        return mx, t3, sums
    return _apply(mx.reshape(BN, cout), t3.reshape(BN, cout), sums,
                  g[None, :], be[None, :], cout, BN * KNN)


def kernel(x, W1, b1, g1, be1, W2, b2, g2, be2, W3, b3, g3, be3, Wf, bf, gf, bef):
    h0 = jnp.transpose(x, (0, 2, 1)).reshape(BN, 3)
    h1 = _edge_layer(h0, 3, 64, W1, b1, g1, be1)
    h2 = _edge_layer(h1, 64, 128, W2, b2, g2, be2)
    mx3, t3, sums3 = _edge_layer(h2, 128, 256, W3, b3, g3, be3, apply_now=False)

    w1 = jnp.transpose(Wf[:, :64])
    w2 = jnp.transpose(Wf[:, 64:192])
    w3 = jnp.transpose(Wf[:, 192:])
    f, sums = _final_mm(h1, h2, mx3.reshape(BN, 256), t3.reshape(BN, 256),
                        sums3, g3[None, :], be3[None, :], w1, w2, w3, bf[None, :])
    return _final_apply(f, sums, gf[None, :], bef[None, :], BN)


# speedup vs baseline: 1.0062x; 1.0062x over previous
import functools

import jax
import jax.numpy as jnp
from jax import lax
from jax.experimental import pallas as pl
from jax.experimental.pallas import tpu as pltpu
from jax.experimental.pallas import tpu_sc as plsc

B = 4
N = 1024
BN = B * N
KNN = 20
EPS = 1e-5
NW = 32
BF16 = jnp.bfloat16


def _topk_body(h_blk_ref, ht_ref, xxc_ref, xxr_ref, idx_ref, *, nb):
    b = pl.program_id(0)
    hb = h_blk_ref[0]
    ht = ht_ref[0]
    d = 2.0 * jnp.dot(hb.astype(BF16), ht.astype(BF16),
                      preferred_element_type=jnp.float32)
    xb = xxc_ref[0]
    xa = xxr_ref[0]
    d = d - xb - xa
    iota_m = lax.broadcasted_iota(jnp.int32, d.shape, 1)
    iota_k = lax.broadcasted_iota(jnp.int32, (nb, KNN), 1)
    out = jnp.zeros((nb, KNN), jnp.int32)
    for j in range(KNN):
        am = jnp.argmax(d, axis=1).astype(jnp.int32)[:, None]
        out = jnp.where(iota_k == j, am, out)
        d = jnp.where(iota_m == am, -jnp.inf, d)
    idx_ref[0] = jnp.transpose(out) + b * N


def _topk(h3d, ht3d, xx, c):
    nb = 512
    body = functools.partial(_topk_body, nb=nb)
    return pl.pallas_call(
        body,
        grid=(B, N // nb),
        in_specs=[
            pl.BlockSpec((1, nb, c), lambda b, i: (b, i, 0)),
            pl.BlockSpec((1, c, N), lambda b, i: (b, 0, 0)),
            pl.BlockSpec((1, nb, 1), lambda b, i: (b, i, 0)),
            pl.BlockSpec((1, 1, N), lambda b, i: (b, 0, 0)),
        ],
        out_specs=pl.BlockSpec((1, KNN, nb), lambda b, i: (b, 0, i)),
        out_shape=jax.ShapeDtypeStruct((B, KNN, N), jnp.int32),
    )(h3d, ht3d, xx.reshape(B, N, 1), xx.reshape(B, 1, N))


def _sc_gather_rows(h2d, idx_flat, c):
    tot = KNN * BN
    rw = tot // NW
    ch = 128
    npair = rw // (2 * ch)
    mesh = plsc.VectorSubcoreMesh(core_axis_name="c", subcore_axis_name="s")

    @functools.partial(
        pl.kernel,
        mesh=mesh,
        compiler_params=pltpu.CompilerParams(use_tc_tiling_on_sc=False),
        out_type=jax.ShapeDtypeStruct((tot, c), jnp.float32),
        scratch_types=[
            pltpu.VMEM((ch,), jnp.int32),
            pltpu.VMEM((ch,), jnp.int32),
            pltpu.VMEM((ch, c), jnp.float32),
            pltpu.VMEM((ch, c), jnp.float32),
            pltpu.SemaphoreType.DMA,
            pltpu.SemaphoreType.DMA,
            pltpu.SemaphoreType.DMA,
            pltpu.SemaphoreType.DMA,
        ],
    )
    def sck(h_hbm, idx_hbm, out_hbm, idx0, idx1, rows0, rows1,
            sem_g, sem_g2, sem_o0, sem_o1):
        wid = lax.axis_index("s") * 2 + lax.axis_index("c")
        base = wid * rw

        def pair(p, carry):
            off0 = base + (2 * p) * ch
            off1 = off0 + ch

            @pl.when(p >= 1)
            def _():
                pltpu.make_async_copy(
                    rows0, out_hbm.at[pl.ds(off0 - 2 * ch, ch)], sem_o0).wait()

            pltpu.sync_copy(idx_hbm.at[pl.ds(off0, ch)], idx0)
            pltpu.async_copy(h_hbm.at[idx0], rows0, sem_g)

            @pl.when(p >= 1)
            def _():
                pltpu.make_async_copy(
                    rows1, out_hbm.at[pl.ds(off1 - 2 * ch, ch)], sem_o1).wait()

            pltpu.sync_copy(idx_hbm.at[pl.ds(off1, ch)], idx1)
            pltpu.async_copy(h_hbm.at[idx1], rows1, sem_g2)
            pltpu.make_async_copy(h_hbm.at[idx0], rows0, sem_g).wait()
            pltpu.async_copy(rows0, out_hbm.at[pl.ds(off0, ch)], sem_o0)
            pltpu.make_async_copy(h_hbm.at[idx1], rows1, sem_g2).wait()
            pltpu.async_copy(rows1, out_hbm.at[pl.ds(off1, ch)], sem_o1)
            return carry

        lax.fori_loop(0, npair, pair, 0, unroll=False)
        end0 = base + (2 * npair - 2) * ch
        pltpu.make_async_copy(rows0, out_hbm.at[pl.ds(end0, ch)], sem_o0).wait()
        pltpu.make_async_copy(rows1, out_hbm.at[pl.ds(end0 + ch, ch)], sem_o1).wait()

    return sck(h2d, idx_flat)


def _econv_body(nbr_ref, h_ref, wa_ref, wb_ref, b_ref, mx_ref, t_ref, sums_ref):
    b = pl.program_id(0)
    i = pl.program_id(1)
    hb = h_ref[0]
    wab = wa_ref[...].astype(BF16)
    s1 = None
    for j in range(KNN):
        diff = nbr_ref[0, j] - hb
        aj = jnp.dot(diff.astype(BF16), wab, preferred_element_type=jnp.float32)
        if s1 is None:
            s1, s2, mx = aj, aj * aj, aj
        else:
            s1 = s1 + aj
            s2 = s2 + aj * aj
            mx = jnp.maximum(mx, aj)
    t = jnp.dot(hb.astype(BF16), wb_ref[...].astype(BF16),
                preferred_element_type=jnp.float32) + b_ref[...]
    mx_ref[0] = mx
    t_ref[0] = t

    @pl.when(jnp.logical_and(b == 0, i == 0))
    def _():
        sums_ref[...] = jnp.zeros_like(sums_ref)

    kf = float(KNN)
    sy = jnp.sum(s1 + kf * t, axis=0)
    sy2 = jnp.sum(s2 + 2.0 * t * s1 + kf * t * t, axis=0)
    sums_ref[0:1, :] += sy[None, :]
    sums_ref[1:2, :] += sy2[None, :]


def _econv(nbr4, h3d, wa, wb, bias, c, cout):
    nb = 256
    return pl.pallas_call(
        _econv_body,
        grid=(B, N // nb),
        in_specs=[
            pl.BlockSpec((1, KNN, nb, c), lambda b, i: (b, 0, i, 0)),
            pl.BlockSpec((1, nb, c), lambda b, i: (b, i, 0)),
            pl.BlockSpec((c, cout), lambda b, i: (0, 0)),
            pl.BlockSpec((c, cout), lambda b, i: (0, 0)),
            pl.BlockSpec((1, cout), lambda b, i: (0, 0)),
        ],
        out_specs=[
            pl.BlockSpec((1, nb, cout), lambda b, i: (b, i, 0)),
            pl.BlockSpec((1, nb, cout), lambda b, i: (b, i, 0)),
            pl.BlockSpec((8, cout), lambda b, i: (0, 0)),
        ],
        out_shape=[
            jax.ShapeDtypeStruct((B, N, cout), jnp.float32),
            jax.ShapeDtypeStruct((B, N, cout), jnp.float32),
            jax.ShapeDtypeStruct((8, cout), jnp.float32),
        ],
    )(nbr4, h3d, wa, wb, bias)


def _apply_body(mx_ref, t_ref, sums_ref, g_ref, be_ref, out_ref, *, count):
    inv_cnt = 1.0 / count
    m = sums_ref[0:1, :] * inv_cnt
    ey2 = sums_ref[1:2, :] * inv_cnt
    v = ey2 - m * m
    denom = jnp.sqrt(v + EPS)
    y = mx_ref[...] + t_ref[...]
    out_ref[...] = jnp.maximum(((y - m) / denom) * g_ref[...] + be_ref[...], 0.0)


def _apply(mx, t, sums, g, be, cout, count):
    mb = 1024
    body = functools.partial(_apply_body, count=count)
    return pl.pallas_call(
        body,
        grid=(BN // mb,),
        in_specs=[
            pl.BlockSpec((mb, cout), lambda i: (i, 0)),
            pl.BlockSpec((mb, cout), lambda i: (i, 0)),
            pl.BlockSpec((8, cout), lambda i: (0, 0)),
            pl.BlockSpec((1, cout), lambda i: (0, 0)),
            pl.BlockSpec((1, cout), lambda i: (0, 0)),
        ],
        out_specs=pl.BlockSpec((mb, cout), lambda i: (i, 0)),
        out_shape=jax.ShapeDtypeStruct((BN, cout), jnp.float32),
    )(mx, t, sums, g, be)


def _final_mm_body(h1_ref, h2_ref, mx3_ref, t3_ref, sums3_ref, g3_ref, be3_ref,
                   w1_ref, w2_ref, w3_ref, b_ref, f_ref, sums_ref, *, count3):
    i = pl.program_id(0)
    inv_cnt = 1.0 / count3
    m3 = sums3_ref[0:1, :] * inv_cnt
    v3 = sums3_ref[1:2, :] * inv_cnt - m3 * m3
    y3 = mx3_ref[...] + t3_ref[...]
    h3 = jnp.maximum(((y3 - m3) / jnp.sqrt(v3 + EPS)) * g3_ref[...]
                     + be3_ref[...], 0.0)
    f = jnp.dot(h1_ref[...].astype(BF16), w1_ref[...].astype(BF16),
                preferred_element_type=jnp.float32)
    f += jnp.dot(h2_ref[...].astype(BF16), w2_ref[...].astype(BF16),
                 preferred_element_type=jnp.float32)
    f += jnp.dot(h3.astype(BF16), w3_ref[...].astype(BF16),
                 preferred_element_type=jnp.float32)
    f = f + b_ref[...]
    f_ref[...] = f

    @pl.when(i == 0)
    def _():
        sums_ref[...] = jnp.zeros_like(sums_ref)

    sums_ref[0:1, :] += jnp.sum(f, axis=0)[None, :]
    sums_ref[1:2, :] += jnp.sum(f * f, axis=0)[None, :]


def _final_mm(h1, h2, mx3, t3, sums3, g3, be3, w1, w2, w3, bias):
    mb = 1024
    body = functools.partial(_final_mm_body, count3=BN * KNN)
    return pl.pallas_call(
        body,
        grid=(BN // mb,),
        in_specs=[
            pl.BlockSpec((mb, 64), lambda i: (i, 0)),
            pl.BlockSpec((mb, 128), lambda i: (i, 0)),
            pl.BlockSpec((mb, 256), lambda i: (i, 0)),
            pl.BlockSpec((mb, 256), lambda i: (i, 0)),
            pl.BlockSpec((8, 256), lambda i: (0, 0)),
            pl.BlockSpec((1, 256), lambda i: (0, 0)),
            pl.BlockSpec((1, 256), lambda i: (0, 0)),
            pl.BlockSpec((64, 512), lambda i: (0, 0)),
            pl.BlockSpec((128, 512), lambda i: (0, 0)),
            pl.BlockSpec((256, 512), lambda i: (0, 0)),
            pl.BlockSpec((1, 512), lambda i: (0, 0)),
        ],
        out_specs=[
            pl.BlockSpec((mb, 512), lambda i: (i, 0)),
            pl.BlockSpec((8, 512), lambda i: (0, 0)),
        ],
        out_shape=[
            jax.ShapeDtypeStruct((BN, 512), jnp.float32),
            jax.ShapeDtypeStruct((8, 512), jnp.float32),
        ],
    )(h1, h2, mx3, t3, sums3, g3, be3, w1, w2, w3, bias)


def _final_apply(f, sums, g, be, count):
    nb = 256

    def body(f_ref, sums_ref, g_ref, be_ref, out_ref):
        inv_cnt = 1.0 / count
        m = sums_ref[0:1, :] * inv_cnt
        ey2 = sums_ref[1:2, :] * inv_cnt
        v = ey2 - m * m
        denom = jnp.sqrt(v + EPS)
        y = jnp.maximum(((f_ref[0] - m) / denom) * g_ref[...] + be_ref[...], 0.0)
        out_ref[0] = jnp.transpose(y)

    return pl.pallas_call(
        body,
        grid=(B, N // nb),
        in_specs=[
            pl.BlockSpec((1, nb, 512), lambda b, i: (b, i, 0)),
            pl.BlockSpec((8, 512), lambda b, i: (0, 0)),
            pl.BlockSpec((1, 512), lambda b, i: (0, 0)),
            pl.BlockSpec((1, 512), lambda b, i: (0, 0)),
        ],
        out_specs=pl.BlockSpec((1, 512, nb), lambda b, i: (b, 0, i)),
        out_shape=jax.ShapeDtypeStruct((B, 512, N), jnp.float32),
    )(f.reshape(B, N, 512), sums, g, be)


def _edge_layer(h2d, c, cout, w, bias, g, be, apply_now=True):
    wa = jnp.transpose(w[:, :c])
    wb = jnp.transpose(w[:, c:])
    cpad = max(c, 16)
    if cpad != c:
        pad = ((0, cpad - c), (0, 0))
        h2d_p = jnp.pad(h2d, ((0, 0), (0, cpad - c)))
        wa = jnp.pad(wa, pad)
        wb = jnp.pad(wb, pad)
    else:
        h2d_p = h2d
    h3d = h2d_p.reshape(B, N, cpad)
    ht3d = jnp.transpose(h3d, (0, 2, 1))
    xx = jnp.sum(ht3d * ht3d, axis=1)
    gidx = _topk(h3d, ht3d, xx, cpad)
    idx_flat = gidx.reshape(B * KNN * N)
    nbr = _sc_gather_rows(h2d_p, idx_flat, cpad)
    nbr4 = nbr.reshape(B, KNN, N, cpad)
    mx, t3, sums = _econv(nbr4, h3d, wa, wb, bias[None, :], cpad, cout)
    if not apply_now:
        return mx, t3, sums
    return _apply(mx.reshape(BN, cout), t3.reshape(BN, cout), sums,
                  g[None, :], be[None, :], cout, BN * KNN)


def kernel(x, W1, b1, g1, be1, W2, b2, g2, be2, W3, b3, g3, be3, Wf, bf, gf, bef):
    h0 = jnp.transpose(x, (0, 2, 1)).reshape(BN, 3)
    h1 = _edge_layer(h0, 3, 64, W1, b1, g1, be1)
    h2 = _edge_layer(h1, 64, 128, W2, b2, g2, be2)
    mx3, t3, sums3 = _edge_layer(h2, 128, 256, W3, b3, g3, be3, apply_now=False)

    w1 = jnp.transpose(Wf[:, :64])
    w2 = jnp.transpose(Wf[:, 64:192])
    w3 = jnp.transpose(Wf[:, 192:])
    f, sums = _final_mm(h1, h2, mx3.reshape(BN, 256), t3.reshape(BN, 256),
                        sums3, g3[None, :], be3[None, :], w1, w2, w3, bf[None, :])
    return _final_apply(f, sums, gf[None, :], bef[None, :], BN)
